# baseline (speedup 1.0000x reference)
import jax
import jax.numpy as jnp
from jax import lax
from jax.experimental import pallas as pl
from jax.experimental.pallas import tpu as pltpu

_BLK = 8192


def _make_body(B):
    def _fused_body(idx_sm, x_ref, c1_ref, y_ref, c2_ref, o_ref):
        b = pl.program_id(0)
        o_ref[...] = x_ref[...] + c1_ref[...]
        base = b * _BLK

        def _patch(k, carry):
            r = idx_sm[k] - base

            @pl.when(jnp.logical_and(r >= 0, r < _BLK))
            def _():
                o_ref[pl.ds(r, 1), :] = (
                    y_ref[pl.ds(k, 1), :] + c2_ref[pl.ds(k, 1), :]
                )

            return carry

        lax.fori_loop(0, B, _patch, 0)

    return _fused_body


def kernel(x, y, c1, c2, index):
    M, D = x.shape
    B = y.shape[0]
    nblk = M // _BLK
    grid_spec = pltpu.PrefetchScalarGridSpec(
        num_scalar_prefetch=1,
        grid=(nblk,),
        in_specs=[
            pl.BlockSpec((_BLK, D), lambda i, *_: (i, 0)),
            pl.BlockSpec((_BLK, D), lambda i, *_: (i, 0)),
            pl.BlockSpec((B, D), lambda i, *_: (0, 0)),
            pl.BlockSpec((B, D), lambda i, *_: (0, 0)),
        ],
        out_specs=pl.BlockSpec((_BLK, D), lambda i, *_: (i, 0)),
    )
    return pl.pallas_call(
        _make_body(B),
        grid_spec=grid_spec,
        out_shape=jax.ShapeDtypeStruct((M, D), x.dtype),
    )(index, x, c1, y, c2)


# speedup vs baseline: 1.1150x; 1.1150x over previous
import jax
import jax.numpy as jnp
from jax import lax
from jax.experimental import pallas as pl
from jax.experimental.pallas import tpu as pltpu

_BLK = 8192


def _make_body(B):
    log2b = max(1, (B - 1).bit_length())

    def _lower_bound(idx_sm, target):
        def _step(_, lohi):
            lo, hi = lohi
            mid = (lo + hi) // 2
            go_right = idx_sm[mid] < target
            return (jnp.where(go_right, mid + 1, lo),
                    jnp.where(go_right, hi, mid))

        lo, _ = lax.fori_loop(
            0, log2b, _step, (jnp.int32(0), jnp.int32(B)))
        return lo

    def _fused_body(idx_sm, x_ref, c1_ref, y_ref, c2_ref, o_ref):
        b = pl.program_id(0)
        o_ref[...] = x_ref[...] + c1_ref[...]
        base = b * _BLK
        lo = _lower_bound(idx_sm, base)
        hi = _lower_bound(idx_sm, base + _BLK)

        def _patch(k, carry):
            r = idx_sm[k] - base
            o_ref[pl.ds(r, 1), :] = (
                y_ref[pl.ds(k, 1), :] + c2_ref[pl.ds(k, 1), :]
            )
            return carry

        lax.fori_loop(lo, hi, _patch, 0)

    return _fused_body


def kernel(x, y, c1, c2, index):
    M, D = x.shape
    B = y.shape[0]
    nblk = M // _BLK
    grid_spec = pltpu.PrefetchScalarGridSpec(
        num_scalar_prefetch=1,
        grid=(nblk,),
        in_specs=[
            pl.BlockSpec((_BLK, D), lambda i, *_: (i, 0)),
            pl.BlockSpec((_BLK, D), lambda i, *_: (i, 0)),
            pl.BlockSpec((B, D), lambda i, *_: (0, 0)),
            pl.BlockSpec((B, D), lambda i, *_: (0, 0)),
        ],
        out_specs=pl.BlockSpec((_BLK, D), lambda i, *_: (i, 0)),
    )
    return pl.pallas_call(
        _make_body(B),
        grid_spec=grid_spec,
        out_shape=jax.ShapeDtypeStruct((M, D), x.dtype),
    )(index, x, c1, y, c2)
